# Initial kernel scaffold; baseline (speedup 1.0000x reference)
import jax
import jax.numpy as jnp
from jax.experimental import pallas as pl


def kernel(x, edge_index, W, b):
    raise NotImplementedError("write your pallas kernel here")


# speedup vs baseline: 30.3426x; 30.3426x over previous
import functools

import jax
import jax.numpy as jnp
from jax import lax
from jax.experimental import pallas as pl
from jax.experimental.pallas import tpu as pltpu, tpu_sc as plsc

_INFO = plsc.get_sparse_core_info()
_NC = _INFO.num_cores
_NS = _INFO.num_subcores
_NW = _NC * _NS

_CH = 125
_ZR = 48


def _row_split(N):
    n0 = (N // _NS) // 8 * 8
    tail = N - n0 * _NS
    assert n0 % _ZR == 0 and tail % 8 == 0 and tail <= _ZR
    return n0, tail


def _make_deg_kernel(N, E):
    rows_per_tile = (E // _CH) // _NW
    n0, tail = _row_split(N)
    mesh = plsc.VectorSubcoreMesh(core_axis_name="c", subcore_axis_name="s")

    @functools.partial(
        pl.kernel,
        out_type=[
            jax.ShapeDtypeStruct((N, 16), jnp.float32),
            jax.ShapeDtypeStruct((N, 16), jnp.float32),
        ],
        mesh=mesh,
        scratch_types=[
            pltpu.VMEM((rows_per_tile, _CH), jnp.int32),
            pltpu.VMEM((_CH, 16), jnp.float32),
            pltpu.VMEM((_ZR, 16), jnp.float32),
            pltpu.VMEM_SHARED((N, 16), jnp.float32),
        ],
    )
    def deg_kernel(dst_hbm, out0, out1, dst_v, ones_v, zero_v, deg_sh):
        cid = lax.axis_index("c")
        sid = lax.axis_index("s")
        wid = cid * _NS + sid

        one16 = jnp.ones((16,), jnp.float32)
        zero16 = jnp.zeros((16,), jnp.float32)
        for r in range(_CH):
            ones_v[r] = one16
        for r in range(_ZR):
            zero_v[r] = zero16

        pltpu.sync_copy(dst_hbm.at[pl.ds(wid * rows_per_tile, rows_per_tile)],
                        dst_v)

        def _zero(k, _):
            pltpu.sync_copy(zero_v, deg_sh.at[pl.ds(sid * n0 + k * _ZR, _ZR)])
            return 0

        lax.fori_loop(0, n0 // _ZR, _zero, 0)

        @pl.when(sid == _NS - 1)
        def _():
            pltpu.sync_copy(zero_v.at[pl.ds(0, tail)],
                            deg_sh.at[pl.ds(_NS * n0, tail)])

        plsc.subcore_barrier()

        def _hist(c, _):
            pltpu.sync_copy(ones_v, deg_sh.at[dst_v.at[c]], add=True)
            return 0

        lax.fori_loop(0, rows_per_tile, _hist, 0)
        plsc.subcore_barrier()

        def _dump(out):
            sl = pl.ds(sid * n0, n0)
            pltpu.sync_copy(deg_sh.at[sl], out.at[sl])

            @pl.when(sid == _NS - 1)
            def _():
                tl = pl.ds(_NS * n0, tail)
                pltpu.sync_copy(deg_sh.at[tl], out.at[tl])

        @pl.when(cid == 0)
        def _():
            _dump(out0)

        @pl.when(cid == 1)
        def _():
            _dump(out1)

    return deg_kernel


def _make_edge_kernel(N, E, D):
    rows_per_tile = (E // _CH) // _NW
    n0, tail = _row_split(N)
    mesh = plsc.VectorSubcoreMesh(core_axis_name="c", subcore_axis_name="s")

    @functools.partial(
        pl.kernel,
        out_type=[
            jax.ShapeDtypeStruct((N, D), jnp.float32),
            jax.ShapeDtypeStruct((N, D), jnp.float32),
        ],
        mesh=mesh,
        scratch_types=[
            pltpu.VMEM((rows_per_tile, _CH), jnp.int32),
            pltpu.VMEM((rows_per_tile, _CH), jnp.int32),
            pltpu.VMEM((_CH, D), jnp.float32),
            pltpu.VMEM((_ZR, D), jnp.float32),
            pltpu.VMEM_SHARED((N, D), jnp.float32),
            pltpu.SemaphoreType.DMA,
        ],
    )
    def edge_kernel(src_hbm, dst_hbm, g_hbm, out0, out1,
                    src_v, dst_v, rows_v, zero_v, acc_sh, sem):
        cid = lax.axis_index("c")
        sid = lax.axis_index("s")
        wid = cid * _NS + sid

        zero16 = jnp.zeros((16,), jnp.float32)
        for r in range(_ZR):
            for l in range(D // 16):
                zero_v[r, pl.ds(l * 16, 16)] = zero16

        pltpu.sync_copy(src_hbm.at[pl.ds(wid * rows_per_tile, rows_per_tile)],
                        src_v)
        pltpu.sync_copy(dst_hbm.at[pl.ds(wid * rows_per_tile, rows_per_tile)],
                        dst_v)

        def _zero(k, _):
            pltpu.sync_copy(zero_v,
                            acc_sh.at[pl.ds(sid * n0 + k * _ZR, _ZR)])
            return 0

        lax.fori_loop(0, n0 // _ZR, _zero, 0)

        @pl.when(sid == _NS - 1)
        def _():
            pltpu.sync_copy(zero_v.at[pl.ds(0, tail)],
                            acc_sh.at[pl.ds(_NS * n0, tail)])

        plsc.subcore_barrier()

        def _edges(c, _):
            pltpu.async_copy(g_hbm.at[src_v.at[c]], rows_v, sem).wait()
            pltpu.sync_copy(rows_v, acc_sh.at[dst_v.at[c]], add=True)
            return 0

        lax.fori_loop(0, rows_per_tile, _edges, 0)
        plsc.subcore_barrier()

        def _dump(out):
            sl = pl.ds(sid * n0, n0)
            pltpu.sync_copy(acc_sh.at[sl], out.at[sl])

            @pl.when(sid == _NS - 1)
            def _():
                tl = pl.ds(_NS * n0, tail)
                pltpu.sync_copy(acc_sh.at[tl], out.at[tl])

        @pl.when(cid == 0)
        def _():
            _dump(out0)

        @pl.when(cid == 1)
        def _():
            _dump(out1)

    return edge_kernel


def _dinv_tc(p0_ref, p1_ref, dinv_ref):
    deg = 1.0 + p0_ref[:, 0:1] + p1_ref[:, 0:1]
    dinv_ref[...] = lax.rsqrt(deg)


def _scale_matmul_tc(x_ref, dinv_ref, w_ref, g_ref):
    xs = x_ref[...] * dinv_ref[...]
    g_ref[...] = jnp.dot(xs, w_ref[...], preferred_element_type=jnp.float32)


def _combine_tc(g_ref, p0_ref, p1_ref, dinv_ref, b_ref, out_ref):
    s = g_ref[...] + p0_ref[...] + p1_ref[...]
    out_ref[...] = dinv_ref[...] * s + b_ref[...]


def kernel(x, edge_index, W, b):
    N, D_in = x.shape
    D_out = W.shape[1]
    E = edge_index.shape[1]

    src_r = edge_index[0].reshape(E // _CH, _CH)
    dst_r = edge_index[1].reshape(E // _CH, _CH)

    hp0, hp1 = _make_deg_kernel(N, E)(dst_r)

    nb = 2000
    grid = (N // nb,)
    dinv = pl.pallas_call(
        _dinv_tc,
        grid=grid,
        in_specs=[
            pl.BlockSpec((nb, 16), lambda i: (i, 0)),
            pl.BlockSpec((nb, 16), lambda i: (i, 0)),
        ],
        out_specs=pl.BlockSpec((nb, 1), lambda i: (i, 0)),
        out_shape=jax.ShapeDtypeStruct((N, 1), jnp.float32),
    )(hp0, hp1)

    g = pl.pallas_call(
        _scale_matmul_tc,
        grid=grid,
        in_specs=[
            pl.BlockSpec((nb, D_in), lambda i: (i, 0)),
            pl.BlockSpec((nb, 1), lambda i: (i, 0)),
            pl.BlockSpec((D_in, D_out), lambda i: (0, 0)),
        ],
        out_specs=pl.BlockSpec((nb, D_out), lambda i: (i, 0)),
        out_shape=jax.ShapeDtypeStruct((N, D_out), jnp.float32),
    )(x, dinv, W)

    p0, p1 = _make_edge_kernel(N, E, D_out)(src_r, dst_r, g)

    out = pl.pallas_call(
        _combine_tc,
        grid=grid,
        in_specs=[
            pl.BlockSpec((nb, D_out), lambda i: (i, 0)),
            pl.BlockSpec((nb, D_out), lambda i: (i, 0)),
            pl.BlockSpec((nb, D_out), lambda i: (i, 0)),
            pl.BlockSpec((nb, 1), lambda i: (i, 0)),
            pl.BlockSpec((1, D_out), lambda i: (0, 0)),
        ],
        out_specs=pl.BlockSpec((nb, D_out), lambda i: (i, 0)),
        out_shape=jax.ShapeDtypeStruct((N, D_out), jnp.float32),
    )(g, p0, p1, dinv, b.reshape(1, D_out))

    return out


# speedup vs baseline: 41.6872x; 1.3739x over previous
import functools

import jax
import jax.numpy as jnp
from jax import lax
from jax.experimental import pallas as pl
from jax.experimental.pallas import tpu as pltpu, tpu_sc as plsc

_INFO = plsc.get_sparse_core_info()
_NC = _INFO.num_cores
_NS = _INFO.num_subcores
_NW = _NC * _NS

_CH = 125
_ZR = 16


def _row_split(N):
    n0 = (N // _NS) // 8 * 8
    tail = N - n0 * _NS
    assert n0 % _ZR == 0 and tail % 8 == 0 and tail <= _ZR
    return n0, tail


def _make_deg_kernel(N, E):
    rows_per_tile = (E // _CH) // _NW
    n0, tail = _row_split(N)
    mesh = plsc.VectorSubcoreMesh(core_axis_name="c", subcore_axis_name="s")

    @functools.partial(
        pl.kernel,
        out_type=[
            jax.ShapeDtypeStruct((N, 16), jnp.float32),
            jax.ShapeDtypeStruct((N, 16), jnp.float32),
        ],
        mesh=mesh,
        scratch_types=[
            pltpu.VMEM((rows_per_tile, _CH), jnp.int32),
            pltpu.VMEM((_CH, 16), jnp.float32),
            pltpu.VMEM((_ZR, 16), jnp.float32),
            pltpu.VMEM_SHARED((N, 16), jnp.float32),
        ],
    )
    def deg_kernel(dst_hbm, out0, out1, dst_v, ones_v, zero_v, deg_sh):
        cid = lax.axis_index("c")
        sid = lax.axis_index("s")
        wid = cid * _NS + sid

        one16 = jnp.ones((16,), jnp.float32)
        zero16 = jnp.zeros((16,), jnp.float32)
        for r in range(_CH):
            ones_v[r] = one16
        for r in range(_ZR):
            zero_v[r] = zero16

        pltpu.sync_copy(dst_hbm.at[pl.ds(wid * rows_per_tile, rows_per_tile)],
                        dst_v)

        def _zero(k, _):
            pltpu.sync_copy(zero_v, deg_sh.at[pl.ds(sid * n0 + k * _ZR, _ZR)])
            return 0

        lax.fori_loop(0, n0 // _ZR, _zero, 0)

        @pl.when(sid == _NS - 1)
        def _():
            pltpu.sync_copy(zero_v.at[pl.ds(0, tail)],
                            deg_sh.at[pl.ds(_NS * n0, tail)])

        plsc.subcore_barrier()

        def _hist(c, _):
            pltpu.sync_copy(ones_v, deg_sh.at[dst_v.at[c]], add=True)
            return 0

        lax.fori_loop(0, rows_per_tile, _hist, 0)
        plsc.subcore_barrier()

        def _dump(out):
            sl = pl.ds(sid * n0, n0)
            pltpu.sync_copy(deg_sh.at[sl], out.at[sl])

            @pl.when(sid == _NS - 1)
            def _():
                tl = pl.ds(_NS * n0, tail)
                pltpu.sync_copy(deg_sh.at[tl], out.at[tl])

        @pl.when(cid == 0)
        def _():
            _dump(out0)

        @pl.when(cid == 1)
        def _():
            _dump(out1)

    return deg_kernel


def _make_edge_kernel(N, E, D):
    rows_per_tile = (E // _CH) // _NW
    half_rows = rows_per_tile // 2
    n0, tail = _row_split(N)
    assert half_rows % 8 == 0 and rows_per_tile % 2 == 0
    mesh = plsc.VectorSubcoreMesh(core_axis_name="c", subcore_axis_name="s")

    @functools.partial(
        pl.kernel,
        out_type=[
            jax.ShapeDtypeStruct((N, D), jnp.float32),
            jax.ShapeDtypeStruct((N, D), jnp.float32),
        ],
        mesh=mesh,
        scratch_types=[
            pltpu.VMEM((half_rows, _CH), jnp.int32),
            pltpu.VMEM((half_rows, _CH), jnp.int32),
            pltpu.VMEM((_CH, D), jnp.float32),
            pltpu.VMEM((_CH, D), jnp.float32),
            pltpu.VMEM((_ZR, D), jnp.float32),
            pltpu.VMEM_SHARED((N, D), jnp.float32),
            pltpu.SemaphoreType.DMA,
            pltpu.SemaphoreType.DMA,
        ],
    )
    def edge_kernel(src_hbm, dst_hbm, g_hbm, out0, out1,
                    src_v, dst_v, rows0_v, rows1_v, zero_v, acc_sh,
                    sem0, sem1):
        cid = lax.axis_index("c")
        sid = lax.axis_index("s")
        wid = cid * _NS + sid

        zero16 = jnp.zeros((16,), jnp.float32)
        for r in range(_ZR):
            for l in range(D // 16):
                zero_v[r, pl.ds(l * 16, 16)] = zero16

        def _zero(k, _):
            pltpu.sync_copy(zero_v,
                            acc_sh.at[pl.ds(sid * n0 + k * _ZR, _ZR)])
            return 0

        lax.fori_loop(0, n0 // _ZR, _zero, 0)

        @pl.when(sid == _NS - 1)
        def _():
            pltpu.sync_copy(zero_v.at[pl.ds(0, tail)],
                            acc_sh.at[pl.ds(_NS * n0, tail)])

        plsc.subcore_barrier()

        def _gather(c, buf, s):
            pltpu.async_copy(g_hbm.at[src_v.at[c]], buf, s)

        def _wait(buf, s):
            pltpu.make_async_copy(g_hbm.at[src_v.at[0]], buf, s).wait()

        def _edges(i, _):
            c0 = 2 * i
            _gather(c0 + 1, rows1_v, sem1)
            _wait(rows0_v, sem0)
            pltpu.sync_copy(rows0_v, acc_sh.at[dst_v.at[c0]], add=True)

            @pl.when(c0 + 2 < half_rows)
            def _():
                _gather(c0 + 2, rows0_v, sem0)

            _wait(rows1_v, sem1)
            pltpu.sync_copy(rows1_v, acc_sh.at[dst_v.at[c0 + 1]], add=True)
            return 0

        for h in range(2):
            base = wid * rows_per_tile + h * half_rows
            pltpu.sync_copy(src_hbm.at[pl.ds(base, half_rows)], src_v)
            pltpu.sync_copy(dst_hbm.at[pl.ds(base, half_rows)], dst_v)
            _gather(0, rows0_v, sem0)
            lax.fori_loop(0, half_rows // 2, _edges, 0)

        plsc.subcore_barrier()

        def _dump(out):
            sl = pl.ds(sid * n0, n0)
            pltpu.sync_copy(acc_sh.at[sl], out.at[sl])

            @pl.when(sid == _NS - 1)
            def _():
                tl = pl.ds(_NS * n0, tail)
                pltpu.sync_copy(acc_sh.at[tl], out.at[tl])

        @pl.when(cid == 0)
        def _():
            _dump(out0)

        @pl.when(cid == 1)
        def _():
            _dump(out1)

    return edge_kernel


def _dinv_matmul_tc(p0_ref, p1_ref, x_ref, w_ref, dinv_ref, g_ref):
    deg = 1.0 + p0_ref[:, 0:1] + p1_ref[:, 0:1]
    dinv = lax.rsqrt(deg)
    dinv_ref[...] = dinv
    xs = x_ref[...] * dinv
    g_ref[...] = jnp.dot(xs, w_ref[...], preferred_element_type=jnp.float32)


def _combine_tc(g_ref, p0_ref, p1_ref, dinv_ref, b_ref, out_ref):
    s = g_ref[...] + p0_ref[...] + p1_ref[...]
    out_ref[...] = dinv_ref[...] * s + b_ref[...]


def kernel(x, edge_index, W, b):
    N, D_in = x.shape
    D_out = W.shape[1]
    E = edge_index.shape[1]

    src_r = edge_index[0].reshape(E // _CH, _CH)
    dst_r = edge_index[1].reshape(E // _CH, _CH)

    hp0, hp1 = _make_deg_kernel(N, E)(dst_r)

    nb = 2000
    grid = (N // nb,)
    dinv, g = pl.pallas_call(
        _dinv_matmul_tc,
        grid=grid,
        in_specs=[
            pl.BlockSpec((nb, 16), lambda i: (i, 0)),
            pl.BlockSpec((nb, 16), lambda i: (i, 0)),
            pl.BlockSpec((nb, D_in), lambda i: (i, 0)),
            pl.BlockSpec((D_in, D_out), lambda i: (0, 0)),
        ],
        out_specs=[
            pl.BlockSpec((nb, 1), lambda i: (i, 0)),
            pl.BlockSpec((nb, D_out), lambda i: (i, 0)),
        ],
        out_shape=[
            jax.ShapeDtypeStruct((N, 1), jnp.float32),
            jax.ShapeDtypeStruct((N, D_out), jnp.float32),
        ],
    )(hp0, hp1, x, W)

    p0, p1 = _make_edge_kernel(N, E, D_out)(src_r, dst_r, g)

    out = pl.pallas_call(
        _combine_tc,
        grid=grid,
        in_specs=[
            pl.BlockSpec((nb, D_out), lambda i: (i, 0)),
            pl.BlockSpec((nb, D_out), lambda i: (i, 0)),
            pl.BlockSpec((nb, D_out), lambda i: (i, 0)),
            pl.BlockSpec((nb, 1), lambda i: (i, 0)),
            pl.BlockSpec((1, D_out), lambda i: (0, 0)),
        ],
        out_specs=pl.BlockSpec((nb, D_out), lambda i: (i, 0)),
        out_shape=jax.ShapeDtypeStruct((N, D_out), jnp.float32),
    )(g, p0, p1, dinv, b.reshape(1, D_out))

    return out
